# baseline (speedup 1.0000x reference)
import jax
import jax.numpy as jnp
from jax.experimental import pallas as pl
from jax.experimental.pallas import tpu as pltpu

_TI = 400


def _gc_body(xt_ref, adj_ref, w_ref, out_ref):
    i = pl.program_id(0)

    @pl.when(i == 0)
    def _init():
        out_ref[...] = jnp.zeros_like(out_ref)

    a_blk = jax.lax.dot_general(
        adj_ref[...], xt_ref[...],
        (((1,), (0,)), ((), ())),
        preferred_element_type=jnp.float32,
    )
    out_ref[...] += jax.lax.dot_general(
        a_blk, w_ref[...],
        (((0,), (0,)), ((), ())),
        preferred_element_type=jnp.float32,
    )


def kernel(x, adj, weight):
    d, n = x.shape
    f = weight.shape[1]
    xt = x.T
    grid = (n // _TI,)
    return pl.pallas_call(
        _gc_body,
        grid=grid,
        in_specs=[
            pl.BlockSpec((n, d), lambda i: (0, 0)),
            pl.BlockSpec((_TI, n), lambda i: (i, 0)),
            pl.BlockSpec((_TI, f), lambda i: (i, 0)),
        ],
        out_specs=pl.BlockSpec((d, f), lambda i: (0, 0)),
        out_shape=jax.ShapeDtypeStruct((d, f), jnp.float32),
        compiler_params=pltpu.CompilerParams(
            dimension_semantics=("arbitrary",),
        ),
    )(xt, adj, weight)
